# baseline (speedup 1.0000x reference)
import functools

import numpy as np
import jax
import jax.numpy as jnp
from jax.experimental import pallas as pl

H = 4
BM = 512
BQ = 512
_EPS = 1e-5


def _lnmm_kernel(x_ref, g_ref, w_ref, o_ref, *, act):
    xb = x_ref[...]
    mu = jnp.mean(xb, axis=-1, keepdims=True)
    var = jnp.mean((xb - mu) ** 2, axis=-1, keepdims=True)
    h = (xb - mu) * jax.lax.rsqrt(var + _EPS) * g_ref[...]
    y = jnp.dot(h, w_ref[...], preferred_element_type=jnp.float32)
    if act == "gelu":
        y = jax.nn.gelu(y)
    o_ref[...] = y


def _lnmm(x, g, w, act=None):
    m, k = x.shape
    n = w.shape[1]
    bm = min(BM, m)
    return pl.pallas_call(
        functools.partial(_lnmm_kernel, act=act),
        grid=(m // bm,),
        in_specs=[
            pl.BlockSpec((bm, k), lambda i: (i, 0)),
            pl.BlockSpec((1, k), lambda i: (0, 0)),
            pl.BlockSpec((k, n), lambda i: (0, 0)),
        ],
        out_specs=pl.BlockSpec((bm, n), lambda i: (i, 0)),
        out_shape=jax.ShapeDtypeStruct((m, n), jnp.float32),
    )(x, g.reshape(1, k), w)


def _mm_res_kernel(x_ref, w_ref, r_ref, o_ref):
    o_ref[...] = r_ref[...] + jnp.dot(
        x_ref[...], w_ref[...], preferred_element_type=jnp.float32)


def _mm_res(x, w, r):
    m, k = x.shape
    n = w.shape[1]
    bm = min(BM, m)
    return pl.pallas_call(
        _mm_res_kernel,
        grid=(m // bm,),
        in_specs=[
            pl.BlockSpec((bm, k), lambda i: (i, 0)),
            pl.BlockSpec((k, n), lambda i: (0, 0)),
            pl.BlockSpec((bm, n), lambda i: (i, 0)),
        ],
        out_specs=pl.BlockSpec((bm, n), lambda i: (i, 0)),
        out_shape=jax.ShapeDtypeStruct((m, n), jnp.float32),
    )(x, w, r)


def _mm_bias_kernel(x_ref, w_ref, b_ref, o_ref):
    o_ref[...] = jnp.dot(
        x_ref[...], w_ref[...], preferred_element_type=jnp.float32) + b_ref[...]


def _mm_bias(x, w, b):
    m, k = x.shape
    n = w.shape[1]
    bm = min(BM, m)
    return pl.pallas_call(
        _mm_bias_kernel,
        grid=(m // bm,),
        in_specs=[
            pl.BlockSpec((bm, k), lambda i: (i, 0)),
            pl.BlockSpec((k, n), lambda i: (0, 0)),
            pl.BlockSpec((1, n), lambda i: (0, 0)),
        ],
        out_specs=pl.BlockSpec((bm, n), lambda i: (i, 0)),
        out_shape=jax.ShapeDtypeStruct((m, n), jnp.float32),
    )(x, w, b.reshape(1, n))


def _mm_bias_res_kernel(x_ref, w_ref, b_ref, r_ref, o_ref):
    o_ref[...] = r_ref[...] + b_ref[...] + jnp.dot(
        x_ref[...], w_ref[...], preferred_element_type=jnp.float32)


def _mm_bias_res(x, w, b, r):
    m, k = x.shape
    n = w.shape[1]
    bm = min(BM, m)
    return pl.pallas_call(
        _mm_bias_res_kernel,
        grid=(m // bm,),
        in_specs=[
            pl.BlockSpec((bm, k), lambda i: (i, 0)),
            pl.BlockSpec((k, n), lambda i: (0, 0)),
            pl.BlockSpec((1, n), lambda i: (0, 0)),
            pl.BlockSpec((bm, n), lambda i: (i, 0)),
        ],
        out_specs=pl.BlockSpec((bm, n), lambda i: (i, 0)),
        out_shape=jax.ShapeDtypeStruct((m, n), jnp.float32),
    )(x, w, b.reshape(1, n), r)


def _attn_kernel(q_ref, k_ref, v_ref, o_ref, *, causal, scale, bq, s):
    i = pl.program_id(1)
    q = q_ref[0]
    k = k_ref[0]
    att = jax.lax.dot_general(
        q, k, (((1,), (1,)), ((), ())),
        preferred_element_type=jnp.float32) * scale
    if causal:
        row = jax.lax.broadcasted_iota(jnp.int32, (bq, s), 0) + i * bq
        col = jax.lax.broadcasted_iota(jnp.int32, (bq, s), 1)
        att = jnp.where(col <= row, att, jnp.float32(-1e9))
    att = att - jnp.max(att, axis=-1, keepdims=True)
    e = jnp.exp(att)
    p = e / jnp.sum(e, axis=-1, keepdims=True)
    o_ref[0] = jnp.dot(p, v_ref[0], preferred_element_type=jnp.float32)


def _attn(q, k, v, causal):
    bh, s, dh = q.shape
    bq = min(BQ, s)
    return pl.pallas_call(
        functools.partial(_attn_kernel, causal=causal,
                          scale=1.0 / float(np.sqrt(dh)), bq=bq, s=s),
        grid=(bh, s // bq),
        in_specs=[
            pl.BlockSpec((1, bq, dh), lambda b, i: (b, i, 0)),
            pl.BlockSpec((1, s, dh), lambda b, i: (b, 0, 0)),
            pl.BlockSpec((1, s, dh), lambda b, i: (b, 0, 0)),
        ],
        out_specs=pl.BlockSpec((1, bq, dh), lambda b, i: (b, i, 0)),
        out_shape=jax.ShapeDtypeStruct((bh, s, dh), jnp.float32),
    )(q, k, v)


def _heads(t, b, s, d):
    dh = d // H
    return t.reshape(b, s, H, dh).transpose(0, 2, 1, 3).reshape(b * H, s, dh)


def _unheads(t, b, s, d):
    dh = d // H
    return t.reshape(b, H, s, dh).transpose(0, 2, 1, 3).reshape(b * s, d)


def _block2d(x2d, p, causal, b, s):
    d = x2d.shape[1]
    h = _lnmm(x2d, p["ln1"], p["Wqkv"])
    q, k, v = jnp.split(h, 3, axis=-1)
    ah = _attn(_heads(q, b, s, d), _heads(k, b, s, d), _heads(v, b, s, d),
               causal)
    a = _unheads(ah, b, s, d)
    x2 = _mm_res(a, p["Wo"], x2d)
    u = _lnmm(x2, p["ln2"], p["W1"], act="gelu")
    return _mm_res(u, p["W2"], x2)


def _combine2d(kv2d, q2d, p, causal, b, s):
    d = q2d.shape[1]
    hq = _lnmm(q2d, p["lnq"], p["Wq"])
    hkv = _lnmm(kv2d, p["lnkv"], p["Wkv"])
    k, v = jnp.split(hkv, 2, axis=-1)
    ah = _attn(_heads(hq, b, s, d), _heads(k, b, s, d), _heads(v, b, s, d),
               causal)
    a = _unheads(ah, b, s, d)
    x2 = _mm_res(a, p["Wo"], q2d)
    u = _lnmm(x2, p["ln2"], p["W1"], act="gelu")
    return _mm_res(u, p["W2"], x2)


def _gate_kernel(x_ref, g_ref, w1_ref, b1_ref, w2_ref, b2_ref, o_ref):
    xb = x_ref[...]
    mu = jnp.mean(xb, axis=-1, keepdims=True)
    var = jnp.mean((xb - mu) ** 2, axis=-1, keepdims=True)
    hln = (xb - mu) * jax.lax.rsqrt(var + _EPS) * g_ref[...]
    h = jnp.maximum(
        jnp.dot(hln, w1_ref[...], preferred_element_type=jnp.float32)
        + b1_ref[...], 0.0)
    o_ref[...] = jnp.dot(
        h, w2_ref[...], preferred_element_type=jnp.float32) + b2_ref[...]


def _gate(x, g, w1, b1, w2, b2):
    m, k = x.shape
    kh = w1.shape[1]
    bm = min(BM, m)
    return pl.pallas_call(
        _gate_kernel,
        grid=(m // bm,),
        in_specs=[
            pl.BlockSpec((bm, k), lambda i: (i, 0)),
            pl.BlockSpec((1, k), lambda i: (0, 0)),
            pl.BlockSpec((k, kh), lambda i: (0, 0)),
            pl.BlockSpec((1, kh), lambda i: (0, 0)),
            pl.BlockSpec((kh, 1), lambda i: (0, 0)),
            pl.BlockSpec((1, 1), lambda i: (0, 0)),
        ],
        out_specs=pl.BlockSpec((bm, 1), lambda i: (i, 0)),
        out_shape=jax.ShapeDtypeStruct((m, 1), jnp.float32),
    )(x, g.reshape(1, k), w1, b1.reshape(1, kh), w2, b2.reshape(1, 1))


def _mask_kernel(gc_ref, gr_ref, o_ref, *, bt, s, si):
    i = pl.program_id(1)
    gi = gc_ref[0]
    gj = gr_ref[0]
    gt = (gj > gi).astype(jnp.float32)
    row = jax.lax.broadcasted_iota(jnp.int32, (bt, s), 0) + i * bt
    col = jax.lax.broadcasted_iota(jnp.int32, (bt, s), 1)
    eq = jnp.logical_and(gj == gi, col < row).astype(jnp.float32)
    rank = jnp.sum(gt + eq, axis=1, keepdims=True)
    o_ref[0] = (rank < si).astype(jnp.float32)


def _topk_mask(gate_b, si):
    b, s = gate_b.shape
    bt = min(BM, s)
    gc = gate_b.reshape(b, s, 1)
    gr = gate_b.reshape(b, 1, s)
    maskc = pl.pallas_call(
        functools.partial(_mask_kernel, bt=bt, s=s, si=si),
        grid=(b, s // bt),
        in_specs=[
            pl.BlockSpec((1, bt, 1), lambda b_, i: (b_, i, 0)),
            pl.BlockSpec((1, 1, s), lambda b_, i: (b_, 0, 0)),
        ],
        out_specs=pl.BlockSpec((1, bt, 1), lambda b_, i: (b_, i, 0)),
        out_shape=jax.ShapeDtypeStruct((b, s, 1), jnp.float32),
    )(gc, gr)
    return maskc.reshape(b * s, 1)


def _ae_kernel(up_ref, m_ref, ew_ref, eb_ref, dw_ref, db_ref,
               sc_ref, acc_ref):
    i = pl.program_id(0)

    @pl.when(i == 0)
    def _init():
        acc_ref[...] = jnp.zeros_like(acc_ref)

    up = up_ref[...]
    mk = m_ref[...]
    z = jnp.dot(up, ew_ref[...], preferred_element_type=jnp.float32) \
        + eb_ref[...]
    hard = (z > 0).astype(jnp.float32)
    ae = jnp.dot(hard, dw_ref[...], preferred_element_type=jnp.float32) \
        + db_ref[...]
    dlt = up - ae
    acc_ref[...] += jnp.sum(dlt * dlt * mk, axis=(0, 1), keepdims=True)
    sc_ref[...] = up * mk


def _ae_scatter(up, maskcol, ew, eb, dw, db):
    m, di = up.shape
    code = ew.shape[1]
    bm = min(BM, m)
    return pl.pallas_call(
        _ae_kernel,
        grid=(m // bm,),
        in_specs=[
            pl.BlockSpec((bm, di), lambda i: (i, 0)),
            pl.BlockSpec((bm, 1), lambda i: (i, 0)),
            pl.BlockSpec((di, code), lambda i: (0, 0)),
            pl.BlockSpec((1, code), lambda i: (0, 0)),
            pl.BlockSpec((code, di), lambda i: (0, 0)),
            pl.BlockSpec((1, di), lambda i: (0, 0)),
        ],
        out_specs=[
            pl.BlockSpec((bm, di), lambda i: (i, 0)),
            pl.BlockSpec((1, 1), lambda i: (0, 0)),
        ],
        out_shape=[
            jax.ShapeDtypeStruct((m, di), jnp.float32),
            jax.ShapeDtypeStruct((1, 1), jnp.float32),
        ],
    )(up, maskcol, ew, eb.reshape(1, code), dw, db.reshape(1, di))


def kernel(x, params):
    p = params
    b, s, d = x.shape
    di = p["up_proj_w"].shape[1]
    si = s // 4
    m = b * s

    x2d = x.reshape(m, d)

    scan = _block2d(x2d, p["up_scan"], False, b, s)
    gate = _gate(scan, p["up_norm_g"], p["up_gate_w1"], p["up_gate_b1"],
                 p["up_gate_w2"], p["up_gate_b2"])
    maskcol = _topk_mask(gate.reshape(b, s), si)

    gathered = _block2d(x2d, p["query_block"], True, b, s)
    up = _mm_bias(gathered, p["up_proj_w"], p["up_proj_b"])

    scattered, acc = _ae_scatter(up, maskcol, p["ae_enc_w"], p["ae_enc_b"],
                                 p["ae_dec_w"], p["ae_dec_b"])
    aux = acc[0, 0] / jnp.float32(b * si * di)

    pos = p["pos_emb_w"] + p["pos_emb_b"][None, :]
    query2d = jnp.broadcast_to(pos[None], (b, s, di)).reshape(m, di)

    pds = _combine2d(scattered, query2d, p["down_scatter"], True, b, s)
    pdsp = _mm_bias(pds, p["down_proj_w"], p["down_proj_b"])

    q2 = _mm_bias_res(query2d, p["down_proj_w"], p["down_proj_b"], x2d)
    p_down = _combine2d(pdsp, q2, p["down_scatter2"], True, b, s)

    out = _block2d(p_down, p["down_scan"], False, b, s)
    return out.reshape(b, s, d), aux


# speedup vs baseline: 1.1132x; 1.1132x over previous
import functools

import numpy as np
import jax
import jax.numpy as jnp
from jax.experimental import pallas as pl

H = 4
BM = 512
BQ = 512
_EPS = 1e-5


def _lnmm_kernel(x_ref, g_ref, w_ref, o_ref, *, act):
    xb = x_ref[...]
    mu = jnp.mean(xb, axis=-1, keepdims=True)
    var = jnp.mean((xb - mu) ** 2, axis=-1, keepdims=True)
    h = (xb - mu) * jax.lax.rsqrt(var + _EPS) * g_ref[...]
    y = jnp.dot(h.astype(w_ref.dtype), w_ref[...],
                preferred_element_type=jnp.float32)
    if act == "gelu":
        y = jax.nn.gelu(y)
    o_ref[...] = y.astype(o_ref.dtype)


def _lnmm(x, g, w, act=None, out_dtype=jnp.float32):
    m, k = x.shape
    n = w.shape[1]
    bm = min(BM, m)
    return pl.pallas_call(
        functools.partial(_lnmm_kernel, act=act),
        grid=(m // bm,),
        in_specs=[
            pl.BlockSpec((bm, k), lambda i: (i, 0)),
            pl.BlockSpec((1, k), lambda i: (0, 0)),
            pl.BlockSpec((k, n), lambda i: (0, 0)),
        ],
        out_specs=pl.BlockSpec((bm, n), lambda i: (i, 0)),
        out_shape=jax.ShapeDtypeStruct((m, n), out_dtype),
    )(x, g.reshape(1, k), w)


def _mm_res_kernel(x_ref, w_ref, r_ref, o_ref):
    o_ref[...] = r_ref[...] + jnp.dot(
        x_ref[...].astype(w_ref.dtype), w_ref[...],
        preferred_element_type=jnp.float32)


def _mm_res(x, w, r):
    m, k = x.shape
    n = w.shape[1]
    bm = min(BM, m)
    return pl.pallas_call(
        _mm_res_kernel,
        grid=(m // bm,),
        in_specs=[
            pl.BlockSpec((bm, k), lambda i: (i, 0)),
            pl.BlockSpec((k, n), lambda i: (0, 0)),
            pl.BlockSpec((bm, n), lambda i: (i, 0)),
        ],
        out_specs=pl.BlockSpec((bm, n), lambda i: (i, 0)),
        out_shape=jax.ShapeDtypeStruct((m, n), jnp.float32),
    )(x, w, r)


def _mm_bias_kernel(x_ref, w_ref, b_ref, o_ref):
    o_ref[...] = jnp.dot(
        x_ref[...].astype(w_ref.dtype), w_ref[...],
        preferred_element_type=jnp.float32) + b_ref[...]


def _mm_bias(x, w, b):
    m, k = x.shape
    n = w.shape[1]
    bm = min(BM, m)
    return pl.pallas_call(
        _mm_bias_kernel,
        grid=(m // bm,),
        in_specs=[
            pl.BlockSpec((bm, k), lambda i: (i, 0)),
            pl.BlockSpec((k, n), lambda i: (0, 0)),
            pl.BlockSpec((1, n), lambda i: (0, 0)),
        ],
        out_specs=pl.BlockSpec((bm, n), lambda i: (i, 0)),
        out_shape=jax.ShapeDtypeStruct((m, n), jnp.float32),
    )(x, w, b.reshape(1, n))


def _mm_bias_res_kernel(x_ref, w_ref, b_ref, r_ref, o_ref):
    o_ref[...] = r_ref[...] + b_ref[...] + jnp.dot(
        x_ref[...].astype(w_ref.dtype), w_ref[...],
        preferred_element_type=jnp.float32)


def _mm_bias_res(x, w, b, r):
    m, k = x.shape
    n = w.shape[1]
    bm = min(BM, m)
    return pl.pallas_call(
        _mm_bias_res_kernel,
        grid=(m // bm,),
        in_specs=[
            pl.BlockSpec((bm, k), lambda i: (i, 0)),
            pl.BlockSpec((k, n), lambda i: (0, 0)),
            pl.BlockSpec((1, n), lambda i: (0, 0)),
            pl.BlockSpec((bm, n), lambda i: (i, 0)),
        ],
        out_specs=pl.BlockSpec((bm, n), lambda i: (i, 0)),
        out_shape=jax.ShapeDtypeStruct((m, n), jnp.float32),
    )(x, w, b.reshape(1, n), r)


def _attn_kernel(q_ref, k_ref, v_ref, o_ref, *, causal, scale, bq, s):
    i = pl.program_id(1)
    q = q_ref[0]
    k = k_ref[0]
    att = jax.lax.dot_general(
        q, k, (((1,), (1,)), ((), ())),
        preferred_element_type=jnp.float32) * scale
    if causal:
        row = jax.lax.broadcasted_iota(jnp.int32, (bq, s), 0) + i * bq
        col = jax.lax.broadcasted_iota(jnp.int32, (bq, s), 1)
        att = jnp.where(col <= row, att, jnp.float32(-1e9))
    att = att - jnp.max(att, axis=-1, keepdims=True)
    e = jnp.exp(att)
    p = e / jnp.sum(e, axis=-1, keepdims=True)
    o_ref[0] = jnp.dot(p.astype(v_ref.dtype), v_ref[0],
                       preferred_element_type=jnp.float32).astype(o_ref.dtype)


def _attn(q, k, v, causal):
    bh, s, dh = q.shape
    bq = min(BQ, s)
    return pl.pallas_call(
        functools.partial(_attn_kernel, causal=causal,
                          scale=1.0 / float(np.sqrt(dh)), bq=bq, s=s),
        grid=(bh, s // bq),
        in_specs=[
            pl.BlockSpec((1, bq, dh), lambda b, i: (b, i, 0)),
            pl.BlockSpec((1, s, dh), lambda b, i: (b, 0, 0)),
            pl.BlockSpec((1, s, dh), lambda b, i: (b, 0, 0)),
        ],
        out_specs=pl.BlockSpec((1, bq, dh), lambda b, i: (b, i, 0)),
        out_shape=jax.ShapeDtypeStruct((bh, s, dh), q.dtype),
    )(q, k, v)


def _heads(t, b, s, d):
    dh = d // H
    return t.reshape(b, s, H, dh).transpose(0, 2, 1, 3).reshape(b * H, s, dh)


def _unheads(t, b, s, d):
    dh = d // H
    return t.reshape(b, H, s, dh).transpose(0, 2, 1, 3).reshape(b * s, d)


def _bf(w):
    return w.astype(jnp.bfloat16)


def _block2d(x2d, p, causal, b, s):
    d = x2d.shape[1]
    h = _lnmm(x2d, p["ln1"], _bf(p["Wqkv"]), out_dtype=jnp.bfloat16)
    q, k, v = jnp.split(h, 3, axis=-1)
    ah = _attn(_heads(q, b, s, d), _heads(k, b, s, d), _heads(v, b, s, d),
               causal)
    a = _unheads(ah, b, s, d)
    x2 = _mm_res(a, _bf(p["Wo"]), x2d)
    u = _lnmm(x2, p["ln2"], _bf(p["W1"]), act="gelu", out_dtype=jnp.bfloat16)
    return _mm_res(u, _bf(p["W2"]), x2)


def _combine2d(kv2d, q2d, p, causal, b, s):
    d = q2d.shape[1]
    hq = _lnmm(q2d, p["lnq"], _bf(p["Wq"]), out_dtype=jnp.bfloat16)
    hkv = _lnmm(kv2d, p["lnkv"], _bf(p["Wkv"]), out_dtype=jnp.bfloat16)
    k, v = jnp.split(hkv, 2, axis=-1)
    ah = _attn(_heads(hq, b, s, d), _heads(k, b, s, d), _heads(v, b, s, d),
               causal)
    a = _unheads(ah, b, s, d)
    x2 = _mm_res(a, _bf(p["Wo"]), q2d)
    u = _lnmm(x2, p["ln2"], _bf(p["W1"]), act="gelu", out_dtype=jnp.bfloat16)
    return _mm_res(u, _bf(p["W2"]), x2)


def _gate_kernel(x_ref, g_ref, w1_ref, b1_ref, w2_ref, b2_ref, o_ref):
    xb = x_ref[...]
    mu = jnp.mean(xb, axis=-1, keepdims=True)
    var = jnp.mean((xb - mu) ** 2, axis=-1, keepdims=True)
    hln = (xb - mu) * jax.lax.rsqrt(var + _EPS) * g_ref[...]
    h = jnp.maximum(
        jnp.dot(hln, w1_ref[...], preferred_element_type=jnp.float32)
        + b1_ref[...], 0.0)
    o_ref[...] = jnp.dot(
        h, w2_ref[...], preferred_element_type=jnp.float32) + b2_ref[...]


def _gate(x, g, w1, b1, w2, b2):
    m, k = x.shape
    kh = w1.shape[1]
    bm = min(BM, m)
    return pl.pallas_call(
        _gate_kernel,
        grid=(m // bm,),
        in_specs=[
            pl.BlockSpec((bm, k), lambda i: (i, 0)),
            pl.BlockSpec((1, k), lambda i: (0, 0)),
            pl.BlockSpec((k, kh), lambda i: (0, 0)),
            pl.BlockSpec((1, kh), lambda i: (0, 0)),
            pl.BlockSpec((kh, 1), lambda i: (0, 0)),
            pl.BlockSpec((1, 1), lambda i: (0, 0)),
        ],
        out_specs=pl.BlockSpec((bm, 1), lambda i: (i, 0)),
        out_shape=jax.ShapeDtypeStruct((m, 1), jnp.float32),
    )(x, g.reshape(1, k), w1, b1.reshape(1, kh), w2, b2.reshape(1, 1))


def _mask_kernel(gc_ref, gr_ref, o_ref, *, bt, s, si):
    i = pl.program_id(1)
    gi = gc_ref[0]
    gj = gr_ref[0]
    gt = (gj > gi).astype(jnp.float32)
    row = jax.lax.broadcasted_iota(jnp.int32, (bt, s), 0) + i * bt
    col = jax.lax.broadcasted_iota(jnp.int32, (bt, s), 1)
    eq = jnp.logical_and(gj == gi, col < row).astype(jnp.float32)
    rank = jnp.sum(gt + eq, axis=1, keepdims=True)
    o_ref[0] = (rank < si).astype(jnp.float32)


def _topk_mask(gate_b, si):
    b, s = gate_b.shape
    bt = min(BM, s)
    gc = gate_b.reshape(b, s, 1)
    gr = gate_b.reshape(b, 1, s)
    maskc = pl.pallas_call(
        functools.partial(_mask_kernel, bt=bt, s=s, si=si),
        grid=(b, s // bt),
        in_specs=[
            pl.BlockSpec((1, bt, 1), lambda b_, i: (b_, i, 0)),
            pl.BlockSpec((1, 1, s), lambda b_, i: (b_, 0, 0)),
        ],
        out_specs=pl.BlockSpec((1, bt, 1), lambda b_, i: (b_, i, 0)),
        out_shape=jax.ShapeDtypeStruct((b, s, 1), jnp.float32),
    )(gc, gr)
    return maskc.reshape(b * s, 1)


def _ae_kernel(up_ref, m_ref, ew_ref, eb_ref, dw_ref, db_ref,
               sc_ref, acc_ref):
    i = pl.program_id(0)

    @pl.when(i == 0)
    def _init():
        acc_ref[...] = jnp.zeros_like(acc_ref)

    up = up_ref[...]
    mk = m_ref[...]
    z = jnp.dot(up.astype(ew_ref.dtype), ew_ref[...],
                preferred_element_type=jnp.float32) + eb_ref[...]
    hard = (z > 0).astype(dw_ref.dtype)
    ae = jnp.dot(hard, dw_ref[...], preferred_element_type=jnp.float32) \
        + db_ref[...]
    dlt = up - ae
    acc_ref[...] += jnp.sum(dlt * dlt * mk, axis=(0, 1), keepdims=True)
    sc_ref[...] = up * mk


def _ae_scatter(up, maskcol, ew, eb, dw, db):
    m, di = up.shape
    code = ew.shape[1]
    bm = min(BM, m)
    return pl.pallas_call(
        _ae_kernel,
        grid=(m // bm,),
        in_specs=[
            pl.BlockSpec((bm, di), lambda i: (i, 0)),
            pl.BlockSpec((bm, 1), lambda i: (i, 0)),
            pl.BlockSpec((di, code), lambda i: (0, 0)),
            pl.BlockSpec((1, code), lambda i: (0, 0)),
            pl.BlockSpec((code, di), lambda i: (0, 0)),
            pl.BlockSpec((1, di), lambda i: (0, 0)),
        ],
        out_specs=[
            pl.BlockSpec((bm, di), lambda i: (i, 0)),
            pl.BlockSpec((1, 1), lambda i: (0, 0)),
        ],
        out_shape=[
            jax.ShapeDtypeStruct((m, di), jnp.float32),
            jax.ShapeDtypeStruct((1, 1), jnp.float32),
        ],
    )(up, maskcol, ew, eb.reshape(1, code), dw, db.reshape(1, di))


def kernel(x, params):
    p = params
    b, s, d = x.shape
    di = p["up_proj_w"].shape[1]
    si = s // 4
    m = b * s

    x2d = x.reshape(m, d)

    scan = _block2d(x2d, p["up_scan"], False, b, s)
    gate = _gate(scan, p["up_norm_g"], p["up_gate_w1"], p["up_gate_b1"],
                 p["up_gate_w2"], p["up_gate_b2"])
    maskcol = _topk_mask(gate.reshape(b, s), si)

    gathered = _block2d(x2d, p["query_block"], True, b, s)
    up = _mm_bias(gathered, _bf(p["up_proj_w"]), p["up_proj_b"])

    scattered, acc = _ae_scatter(up, maskcol, _bf(p["ae_enc_w"]),
                                 p["ae_enc_b"], _bf(p["ae_dec_w"]),
                                 p["ae_dec_b"])
    aux = acc[0, 0] / jnp.float32(b * si * di)

    pos = p["pos_emb_w"] + p["pos_emb_b"][None, :]
    query2d = jnp.broadcast_to(pos[None], (b, s, di)).reshape(m, di)

    pds = _combine2d(scattered, query2d, p["down_scatter"], True, b, s)
    pdsp = _mm_bias(pds, _bf(p["down_proj_w"]), p["down_proj_b"])

    q2 = _mm_bias_res(query2d, _bf(p["down_proj_w"]), p["down_proj_b"], x2d)
    p_down = _combine2d(pdsp, q2, p["down_scatter2"], True, b, s)

    out = _block2d(p_down, p["down_scan"], False, b, s)
    return out.reshape(b, s, d), aux


# speedup vs baseline: 1.5818x; 1.4210x over previous
import functools

import numpy as np
import jax
import jax.numpy as jnp
from jax.experimental import pallas as pl

H = 4
BM = 512
BQ = 512
_EPS = 1e-5
_BF = jnp.bfloat16


def _ln_f32(xb, g):
    mu = jnp.mean(xb, axis=-1, keepdims=True)
    var = jnp.mean((xb - mu) ** 2, axis=-1, keepdims=True)
    return (xb - mu) * jax.lax.rsqrt(var + _EPS) * g


def _qkv_kernel(x_ref, g_ref, w_ref, o_ref, *, nj):
    h = _ln_f32(x_ref[...], g_ref[...]).astype(_BF)
    for j in range(nj):
        o_ref[j] = jnp.dot(h, w_ref[j],
                           preferred_element_type=jnp.float32).astype(o_ref.dtype)


def _qkv(x, g, w_r):
    m, k = x.shape
    nj, _, dh = w_r.shape
    bm = min(BM, m)
    return pl.pallas_call(
        functools.partial(_qkv_kernel, nj=nj),
        grid=(m // bm,),
        in_specs=[
            pl.BlockSpec((bm, k), lambda i: (i, 0)),
            pl.BlockSpec((1, k), lambda i: (0, 0)),
            pl.BlockSpec((nj, k, dh), lambda i: (0, 0, 0)),
        ],
        out_specs=pl.BlockSpec((nj, bm, dh), lambda i: (0, i, 0)),
        out_shape=jax.ShapeDtypeStruct((nj, m, dh), _BF),
    )(x, g.reshape(1, k), w_r)


def _attn_kernel(q_ref, k_ref, v_ref, o_ref, *, causal, scale, bq, s):
    i = pl.program_id(2)
    att = jax.lax.dot_general(
        q_ref[0], k_ref[0], (((1,), (1,)), ((), ())),
        preferred_element_type=jnp.float32) * scale
    if causal:
        row = jax.lax.broadcasted_iota(jnp.int32, (bq, s), 0) + i * bq
        col = jax.lax.broadcasted_iota(jnp.int32, (bq, s), 1)
        att = jnp.where(col <= row, att, jnp.float32(-1e9))
    att = att - jnp.max(att, axis=-1, keepdims=True)
    e = jnp.exp(att)
    p = e / jnp.sum(e, axis=-1, keepdims=True)
    o_ref[0] = jnp.dot(p.astype(v_ref.dtype), v_ref[0],
                       preferred_element_type=jnp.float32).astype(o_ref.dtype)


def _attn_hm(qa, ka, ko, vo, causal, b, s):
    _, m, dh = qa.shape
    bq = min(BQ, s)
    nb = s // bq
    return pl.pallas_call(
        functools.partial(_attn_kernel, causal=causal,
                          scale=1.0 / float(np.sqrt(dh)), bq=bq, s=s),
        grid=(b, H, nb),
        in_specs=[
            pl.BlockSpec((1, bq, dh),
                         lambda bi, h, i, nb=nb: (h, bi * nb + i, 0)),
            pl.BlockSpec((1, s, dh), lambda bi, h, i, ko=ko: (ko + h, bi, 0)),
            pl.BlockSpec((1, s, dh), lambda bi, h, i, vo=vo: (vo + h, bi, 0)),
        ],
        out_specs=pl.BlockSpec((1, bq, dh),
                               lambda bi, h, i, nb=nb: (h, bi * nb + i, 0)),
        out_shape=jax.ShapeDtypeStruct((H, m, dh), _BF),
    )(qa, ka, ka)


def _wo_res_kernel(a_ref, w_ref, r_ref, o_ref, *, nh):
    acc = r_ref[...]
    for h in range(nh):
        acc = acc + jnp.dot(a_ref[h], w_ref[h],
                            preferred_element_type=jnp.float32)
    o_ref[...] = acc


def _wo_res(a_hm, w_r, r):
    nh, m, dh = a_hm.shape
    n = w_r.shape[2]
    bm = min(BM, m)
    return pl.pallas_call(
        functools.partial(_wo_res_kernel, nh=nh),
        grid=(m // bm,),
        in_specs=[
            pl.BlockSpec((nh, bm, dh), lambda i: (0, i, 0)),
            pl.BlockSpec((nh, dh, n), lambda i: (0, 0, 0)),
            pl.BlockSpec((bm, n), lambda i: (i, 0)),
        ],
        out_specs=pl.BlockSpec((bm, n), lambda i: (i, 0)),
        out_shape=jax.ShapeDtypeStruct((m, n), jnp.float32),
    )(a_hm, w_r, r)


def _lnmm_kernel(x_ref, g_ref, w_ref, o_ref, *, act):
    h = _ln_f32(x_ref[...], g_ref[...])
    y = jnp.dot(h.astype(w_ref.dtype), w_ref[...],
                preferred_element_type=jnp.float32)
    if act == "gelu":
        y = jax.nn.gelu(y)
    o_ref[...] = y.astype(o_ref.dtype)


def _lnmm(x, g, w, act=None, out_dtype=jnp.float32):
    m, k = x.shape
    n = w.shape[1]
    bm = min(BM, m)
    return pl.pallas_call(
        functools.partial(_lnmm_kernel, act=act),
        grid=(m // bm,),
        in_specs=[
            pl.BlockSpec((bm, k), lambda i: (i, 0)),
            pl.BlockSpec((1, k), lambda i: (0, 0)),
            pl.BlockSpec((k, n), lambda i: (0, 0)),
        ],
        out_specs=pl.BlockSpec((bm, n), lambda i: (i, 0)),
        out_shape=jax.ShapeDtypeStruct((m, n), out_dtype),
    )(x, g.reshape(1, k), w)


def _mm_res_kernel(x_ref, w_ref, r_ref, o_ref):
    o_ref[...] = r_ref[...] + jnp.dot(
        x_ref[...].astype(w_ref.dtype), w_ref[...],
        preferred_element_type=jnp.float32)


def _mm_res(x, w, r):
    m, k = x.shape
    n = w.shape[1]
    bm = min(BM, m)
    return pl.pallas_call(
        _mm_res_kernel,
        grid=(m // bm,),
        in_specs=[
            pl.BlockSpec((bm, k), lambda i: (i, 0)),
            pl.BlockSpec((k, n), lambda i: (0, 0)),
            pl.BlockSpec((bm, n), lambda i: (i, 0)),
        ],
        out_specs=pl.BlockSpec((bm, n), lambda i: (i, 0)),
        out_shape=jax.ShapeDtypeStruct((m, n), jnp.float32),
    )(x, w, r)


def _mm_res_proj_kernel(x_ref, w_ref, r_ref, pw_ref, pb_ref, o_ref):
    t = r_ref[...] + jnp.dot(x_ref[...].astype(w_ref.dtype), w_ref[...],
                             preferred_element_type=jnp.float32)
    o_ref[...] = jnp.dot(t.astype(pw_ref.dtype), pw_ref[...],
                         preferred_element_type=jnp.float32) + pb_ref[...]


def _mm_res_proj(x, w, r, pw, pb):
    m, k = x.shape
    n = w.shape[1]
    n2 = pw.shape[1]
    bm = min(BM, m)
    return pl.pallas_call(
        _mm_res_proj_kernel,
        grid=(m // bm,),
        in_specs=[
            pl.BlockSpec((bm, k), lambda i: (i, 0)),
            pl.BlockSpec((k, n), lambda i: (0, 0)),
            pl.BlockSpec((bm, n), lambda i: (i, 0)),
            pl.BlockSpec((n, n2), lambda i: (0, 0)),
            pl.BlockSpec((1, n2), lambda i: (0, 0)),
        ],
        out_specs=pl.BlockSpec((bm, n2), lambda i: (i, 0)),
        out_shape=jax.ShapeDtypeStruct((m, n2), jnp.float32),
    )(x, w, r, pw, pb.reshape(1, n2))


def _mm_bias_res_kernel(x_ref, w_ref, b_ref, r_ref, o_ref):
    o_ref[...] = r_ref[...] + b_ref[...] + jnp.dot(
        x_ref[...].astype(w_ref.dtype), w_ref[...],
        preferred_element_type=jnp.float32)


def _mm_bias_res(x, w, b, r):
    m, k = x.shape
    n = w.shape[1]
    bm = min(BM, m)
    return pl.pallas_call(
        _mm_bias_res_kernel,
        grid=(m // bm,),
        in_specs=[
            pl.BlockSpec((bm, k), lambda i: (i, 0)),
            pl.BlockSpec((k, n), lambda i: (0, 0)),
            pl.BlockSpec((1, n), lambda i: (0, 0)),
            pl.BlockSpec((bm, n), lambda i: (i, 0)),
        ],
        out_specs=pl.BlockSpec((bm, n), lambda i: (i, 0)),
        out_shape=jax.ShapeDtypeStruct((m, n), jnp.float32),
    )(x, w, b.reshape(1, n), r)


def _w2_gate_kernel(u_ref, w_ref, r_ref, g_ref, w1_ref, b1_ref, w2_ref,
                    b2_ref, o_ref):
    scan = r_ref[...] + jnp.dot(u_ref[...].astype(w_ref.dtype), w_ref[...],
                                preferred_element_type=jnp.float32)
    hln = _ln_f32(scan, g_ref[...])
    hmid = jnp.maximum(
        jnp.dot(hln, w1_ref[...], preferred_element_type=jnp.float32)
        + b1_ref[...], 0.0)
    o_ref[...] = jnp.dot(hmid, w2_ref[...],
                         preferred_element_type=jnp.float32) + b2_ref[...]


def _w2_gate(u, w, r, g, w1, b1, w2, b2):
    m, k = u.shape
    n = w.shape[1]
    kh = w1.shape[1]
    bm = min(BM, m)
    return pl.pallas_call(
        _w2_gate_kernel,
        grid=(m // bm,),
        in_specs=[
            pl.BlockSpec((bm, k), lambda i: (i, 0)),
            pl.BlockSpec((k, n), lambda i: (0, 0)),
            pl.BlockSpec((bm, n), lambda i: (i, 0)),
            pl.BlockSpec((1, n), lambda i: (0, 0)),
            pl.BlockSpec((n, kh), lambda i: (0, 0)),
            pl.BlockSpec((1, kh), lambda i: (0, 0)),
            pl.BlockSpec((kh, 1), lambda i: (0, 0)),
            pl.BlockSpec((1, 1), lambda i: (0, 0)),
        ],
        out_specs=pl.BlockSpec((bm, 1), lambda i: (i, 0)),
        out_shape=jax.ShapeDtypeStruct((m, 1), jnp.float32),
    )(u, w, r, g.reshape(1, n), w1, b1.reshape(1, kh), w2, b2.reshape(1, 1))


def _mask_kernel(gc_ref, gr_ref, o_ref, *, bt, s, si):
    i = pl.program_id(1)
    gi = gc_ref[0]
    gj = gr_ref[0]
    gt = (gj > gi).astype(jnp.float32)
    row = jax.lax.broadcasted_iota(jnp.int32, (bt, s), 0) + i * bt
    col = jax.lax.broadcasted_iota(jnp.int32, (bt, s), 1)
    eq = jnp.logical_and(gj == gi, col < row).astype(jnp.float32)
    rank = jnp.sum(gt + eq, axis=1, keepdims=True)
    o_ref[0] = (rank < si).astype(jnp.float32)


def _topk_mask(gate_b, si):
    b, s = gate_b.shape
    bt = min(BM, s)
    gc = gate_b.reshape(b, s, 1)
    gr = gate_b.reshape(b, 1, s)
    maskc = pl.pallas_call(
        functools.partial(_mask_kernel, bt=bt, s=s, si=si),
        grid=(b, s // bt),
        in_specs=[
            pl.BlockSpec((1, bt, 1), lambda b_, i: (b_, i, 0)),
            pl.BlockSpec((1, 1, s), lambda b_, i: (b_, 0, 0)),
        ],
        out_specs=pl.BlockSpec((1, bt, 1), lambda b_, i: (b_, i, 0)),
        out_shape=jax.ShapeDtypeStruct((b, s, 1), jnp.float32),
    )(gc, gr)
    return maskc.reshape(b * s, 1)


def _up_ae_kernel(x_ref, uw_ref, ub_ref, m_ref, ew_ref, eb_ref, dw_ref,
                  db_ref, sc_ref, acc_ref):
    i = pl.program_id(0)

    @pl.when(i == 0)
    def _init():
        acc_ref[...] = jnp.zeros_like(acc_ref)

    up = jnp.dot(x_ref[...].astype(uw_ref.dtype), uw_ref[...],
                 preferred_element_type=jnp.float32) + ub_ref[...]
    mk = m_ref[...]
    z = jnp.dot(up.astype(ew_ref.dtype), ew_ref[...],
                preferred_element_type=jnp.float32) + eb_ref[...]
    hard = (z > 0).astype(dw_ref.dtype)
    ae = jnp.dot(hard, dw_ref[...], preferred_element_type=jnp.float32) \
        + db_ref[...]
    dlt = up - ae
    acc_ref[...] += jnp.sum(dlt * dlt * mk, axis=(0, 1), keepdims=True)
    sc_ref[...] = up * mk


def _up_ae_scatter(x, uw, ub, maskcol, ew, eb, dw, db):
    m, d = x.shape
    di = uw.shape[1]
    code = ew.shape[1]
    bm = min(BM, m)
    return pl.pallas_call(
        _up_ae_kernel,
        grid=(m // bm,),
        in_specs=[
            pl.BlockSpec((bm, d), lambda i: (i, 0)),
            pl.BlockSpec((d, di), lambda i: (0, 0)),
            pl.BlockSpec((1, di), lambda i: (0, 0)),
            pl.BlockSpec((bm, 1), lambda i: (i, 0)),
            pl.BlockSpec((di, code), lambda i: (0, 0)),
            pl.BlockSpec((1, code), lambda i: (0, 0)),
            pl.BlockSpec((code, di), lambda i: (0, 0)),
            pl.BlockSpec((1, di), lambda i: (0, 0)),
        ],
        out_specs=[
            pl.BlockSpec((bm, di), lambda i: (i, 0)),
            pl.BlockSpec((1, 1), lambda i: (0, 0)),
        ],
        out_shape=[
            jax.ShapeDtypeStruct((m, di), jnp.float32),
            jax.ShapeDtypeStruct((1, 1), jnp.float32),
        ],
    )(x, uw, ub.reshape(1, di), maskcol, ew, eb.reshape(1, code), dw,
      db.reshape(1, di))


def _bf(w):
    return w.astype(_BF)


def _qkv_weights(w, d, groups):
    dh = d // H
    return _bf(w.reshape(d, groups, H, dh).transpose(1, 2, 0, 3)
               .reshape(groups * H, d, dh))


def _wo_weights(w, d):
    dh = d // H
    return _bf(w.reshape(H, dh, w.shape[1]))


def _block2d(x2d, p, causal, b, s):
    d = x2d.shape[1]
    qkv = _qkv(x2d, p["ln1"], _qkv_weights(p["Wqkv"], d, 3))
    ah = _attn_hm(qkv, qkv, H, 2 * H, causal, b, s)
    x2 = _wo_res(ah, _wo_weights(p["Wo"], d), x2d)
    u = _lnmm(x2, p["ln2"], _bf(p["W1"]), act="gelu", out_dtype=_BF)
    return x2, u


def _block(x2d, p, causal, b, s):
    x2, u = _block2d(x2d, p, causal, b, s)
    return _mm_res(u, _bf(p["W2"]), x2)


def _combine_parts(kv2d, q2d, p, causal, b, s):
    d = q2d.shape[1]
    hq = _qkv(q2d, p["lnq"], _qkv_weights(p["Wq"], d, 1))
    hkv = _qkv(kv2d, p["lnkv"], _qkv_weights(p["Wkv"], d, 2))
    ah = _attn_hm(hq, hkv, 0, H, causal, b, s)
    x2 = _wo_res(ah, _wo_weights(p["Wo"], d), q2d)
    u = _lnmm(x2, p["ln2"], _bf(p["W1"]), act="gelu", out_dtype=_BF)
    return x2, u


def _combine(kv2d, q2d, p, causal, b, s):
    x2, u = _combine_parts(kv2d, q2d, p, causal, b, s)
    return _mm_res(u, _bf(p["W2"]), x2)


def kernel(x, params):
    p = params
    b, s, d = x.shape
    di = p["up_proj_w"].shape[1]
    si = s // 4
    m = b * s

    x2d = x.reshape(m, d)

    x2u, uu = _block2d(x2d, p["up_scan"], False, b, s)
    gate = _w2_gate(uu, _bf(p["up_scan"]["W2"]), x2u, p["up_norm_g"],
                    p["up_gate_w1"], p["up_gate_b1"],
                    p["up_gate_w2"], p["up_gate_b2"])
    maskcol = _topk_mask(gate.reshape(b, s), si)

    gathered = _block(x2d, p["query_block"], True, b, s)
    scattered, acc = _up_ae_scatter(
        gathered, _bf(p["up_proj_w"]), p["up_proj_b"], maskcol,
        _bf(p["ae_enc_w"]), p["ae_enc_b"], _bf(p["ae_dec_w"]),
        p["ae_dec_b"])
    aux = acc[0, 0] / jnp.float32(b * si * di)

    pos = p["pos_emb_w"] + p["pos_emb_b"][None, :]
    query2d = jnp.broadcast_to(pos[None], (b, s, di)).reshape(m, di)

    x2c, uc = _combine_parts(scattered, query2d, p["down_scatter"], True,
                             b, s)
    pdsp = _mm_res_proj(uc, _bf(p["down_scatter"]["W2"]), x2c,
                        _bf(p["down_proj_w"]), p["down_proj_b"])

    q2 = _mm_bias_res(query2d, _bf(p["down_proj_w"]), p["down_proj_b"], x2d)
    p_down = _combine(pdsp, q2, p["down_scatter2"], True, b, s)

    out = _block(p_down, p["down_scan"], False, b, s)
    return out.reshape(b, s, d), aux


# speedup vs baseline: 1.6480x; 1.0419x over previous
import functools

import numpy as np
import jax
import jax.numpy as jnp
from jax.experimental import pallas as pl

H = 4
BM = 512
BQ = 512
_EPS = 1e-5
_BF = jnp.bfloat16


def _ln_f32(xb, g):
    mu = jnp.mean(xb, axis=-1, keepdims=True)
    var = jnp.mean((xb - mu) ** 2, axis=-1, keepdims=True)
    return (xb - mu) * jax.lax.rsqrt(var + _EPS) * g


def _qkv_kernel(x_ref, g_ref, w_ref, o_ref, *, nj):
    h = _ln_f32(x_ref[...], g_ref[...]).astype(_BF)
    for j in range(nj):
        o_ref[j] = jnp.dot(h, w_ref[j],
                           preferred_element_type=jnp.float32).astype(o_ref.dtype)


def _qkv(x, g, w_r):
    m, k = x.shape
    nj, _, dh = w_r.shape
    bm = min(BM, m)
    return pl.pallas_call(
        functools.partial(_qkv_kernel, nj=nj),
        grid=(m // bm,),
        in_specs=[
            pl.BlockSpec((bm, k), lambda i: (i, 0)),
            pl.BlockSpec((1, k), lambda i: (0, 0)),
            pl.BlockSpec((nj, k, dh), lambda i: (0, 0, 0)),
        ],
        out_specs=pl.BlockSpec((nj, bm, dh), lambda i: (0, i, 0)),
        out_shape=jax.ShapeDtypeStruct((nj, m, dh), _BF),
    )(x, g.reshape(1, k), w_r)


def _attn_kernel(q_ref, k_ref, v_ref, o_ref, *, causal, scale, bq, s):
    i = pl.program_id(2)
    att = jax.lax.dot_general(
        q_ref[0], k_ref[0], (((1,), (1,)), ((), ())),
        preferred_element_type=jnp.float32) * scale
    if causal:
        row = jax.lax.broadcasted_iota(jnp.int32, (bq, s), 0) + i * bq
        col = jax.lax.broadcasted_iota(jnp.int32, (bq, s), 1)
        att = jnp.where(col <= row, att, jnp.float32(-1e9))
    att = att - jnp.max(att, axis=-1, keepdims=True)
    e = jnp.exp(att)
    p = e / jnp.sum(e, axis=-1, keepdims=True)
    o_ref[0] = jnp.dot(p.astype(v_ref.dtype), v_ref[0],
                       preferred_element_type=jnp.float32).astype(o_ref.dtype)


def _attn_hm(qa, ka, ko, vo, causal, b, s):
    _, m, dh = qa.shape
    bq = min(BQ, s)
    nb = s // bq
    return pl.pallas_call(
        functools.partial(_attn_kernel, causal=causal,
                          scale=1.0 / float(np.sqrt(dh)), bq=bq, s=s),
        grid=(b, H, nb),
        in_specs=[
            pl.BlockSpec((1, bq, dh),
                         lambda bi, h, i, nb=nb: (h, bi * nb + i, 0)),
            pl.BlockSpec((1, s, dh), lambda bi, h, i, ko=ko: (ko + h, bi, 0)),
            pl.BlockSpec((1, s, dh), lambda bi, h, i, vo=vo: (vo + h, bi, 0)),
        ],
        out_specs=pl.BlockSpec((1, bq, dh),
                               lambda bi, h, i, nb=nb: (h, bi * nb + i, 0)),
        out_shape=jax.ShapeDtypeStruct((H, m, dh), _BF),
    )(qa, ka, ka)


def _attn_causal_kernel(q_ref, k_ref, v_ref, o_ref, *, scale, bq, dh):
    j = pl.program_id(2)
    q = q_ref[0, 0]
    row = jax.lax.broadcasted_iota(jnp.int32, (bq, bq), 0)
    col = jax.lax.broadcasted_iota(jnp.int32, (bq, bq), 1)

    def body(t, carry):
        m_prev, l_prev, acc = carry
        kb = k_ref[0, 0, pl.ds(t * bq, bq), :]
        vb = v_ref[0, 0, pl.ds(t * bq, bq), :]
        s_ = jax.lax.dot_general(
            q, kb, (((1,), (1,)), ((), ())),
            preferred_element_type=jnp.float32) * scale
        s_ = jnp.where(t * bq + col <= j * bq + row, s_, jnp.float32(-1e9))
        m_new = jnp.maximum(m_prev, jnp.max(s_, axis=-1, keepdims=True))
        p = jnp.exp(s_ - m_new)
        corr = jnp.exp(m_prev - m_new)
        l_new = l_prev * corr + jnp.sum(p, axis=-1, keepdims=True)
        acc_new = acc * corr + jnp.dot(p.astype(v_ref.dtype), vb,
                                       preferred_element_type=jnp.float32)
        return m_new, l_new, acc_new

    carry0 = (jnp.full((bq, 1), -1e30, jnp.float32),
              jnp.zeros((bq, 1), jnp.float32),
              jnp.zeros((bq, dh), jnp.float32))
    _, l_f, acc = jax.lax.fori_loop(0, j + 1, body, carry0)
    o_ref[0, 0] = (acc / l_f).astype(o_ref.dtype)


def _attn_causal_hm(qa, ka, ko, vo, b, s, q_shared=False):
    njq, _, dh = qa.shape
    bq = min(BQ, s)
    nb = s // bq
    qb = 1 if q_shared else b
    qa4 = qa.reshape(njq, qb, s, dh)
    ka4 = ka.reshape(ka.shape[0], b, s, dh)
    out = pl.pallas_call(
        functools.partial(_attn_causal_kernel,
                          scale=1.0 / float(np.sqrt(dh)), bq=bq, dh=dh),
        grid=(b, H, nb),
        in_specs=[
            pl.BlockSpec((1, 1, bq, dh),
                         lambda bi, h, jj, shared=q_shared:
                         (h, 0 if shared else bi, jj, 0)),
            pl.BlockSpec((1, 1, s, dh),
                         lambda bi, h, jj, ko=ko: (ko + h, bi, 0, 0)),
            pl.BlockSpec((1, 1, s, dh),
                         lambda bi, h, jj, vo=vo: (vo + h, bi, 0, 0)),
        ],
        out_specs=pl.BlockSpec((1, 1, bq, dh),
                               lambda bi, h, jj: (h, bi, jj, 0)),
        out_shape=jax.ShapeDtypeStruct((H, b, s, dh), _BF),
    )(qa4, ka4, ka4)
    return out.reshape(H, b * s, dh)


def _wo_res_kernel(a_ref, w_ref, r_ref, o_ref, *, nh):
    acc = r_ref[...]
    for h in range(nh):
        acc = acc + jnp.dot(a_ref[h], w_ref[h],
                            preferred_element_type=jnp.float32)
    o_ref[...] = acc


def _wo_res(a_hm, w_r, r, r_wrap=None):
    nh, m, dh = a_hm.shape
    n = w_r.shape[2]
    bm = min(BM, m)
    if r_wrap is None:
        r_spec = pl.BlockSpec((bm, n), lambda i: (i, 0))
    else:
        r_spec = pl.BlockSpec((bm, n), lambda i, w=r_wrap: (i % w, 0))
    return pl.pallas_call(
        functools.partial(_wo_res_kernel, nh=nh),
        grid=(m // bm,),
        in_specs=[
            pl.BlockSpec((nh, bm, dh), lambda i: (0, i, 0)),
            pl.BlockSpec((nh, dh, n), lambda i: (0, 0, 0)),
            r_spec,
        ],
        out_specs=pl.BlockSpec((bm, n), lambda i: (i, 0)),
        out_shape=jax.ShapeDtypeStruct((m, n), jnp.float32),
    )(a_hm, w_r, r)


def _lnmm_kernel(x_ref, g_ref, w_ref, o_ref, *, act):
    h = _ln_f32(x_ref[...], g_ref[...])
    y = jnp.dot(h.astype(w_ref.dtype), w_ref[...],
                preferred_element_type=jnp.float32)
    if act == "gelu":
        y = jax.nn.gelu(y)
    o_ref[...] = y.astype(o_ref.dtype)


def _lnmm(x, g, w, act=None, out_dtype=jnp.float32):
    m, k = x.shape
    n = w.shape[1]
    bm = min(BM, m)
    return pl.pallas_call(
        functools.partial(_lnmm_kernel, act=act),
        grid=(m // bm,),
        in_specs=[
            pl.BlockSpec((bm, k), lambda i: (i, 0)),
            pl.BlockSpec((1, k), lambda i: (0, 0)),
            pl.BlockSpec((k, n), lambda i: (0, 0)),
        ],
        out_specs=pl.BlockSpec((bm, n), lambda i: (i, 0)),
        out_shape=jax.ShapeDtypeStruct((m, n), out_dtype),
    )(x, g.reshape(1, k), w)


def _mm_res_kernel(x_ref, w_ref, r_ref, o_ref):
    o_ref[...] = r_ref[...] + jnp.dot(
        x_ref[...].astype(w_ref.dtype), w_ref[...],
        preferred_element_type=jnp.float32)


def _mm_res(x, w, r):
    m, k = x.shape
    n = w.shape[1]
    bm = min(BM, m)
    return pl.pallas_call(
        _mm_res_kernel,
        grid=(m // bm,),
        in_specs=[
            pl.BlockSpec((bm, k), lambda i: (i, 0)),
            pl.BlockSpec((k, n), lambda i: (0, 0)),
            pl.BlockSpec((bm, n), lambda i: (i, 0)),
        ],
        out_specs=pl.BlockSpec((bm, n), lambda i: (i, 0)),
        out_shape=jax.ShapeDtypeStruct((m, n), jnp.float32),
    )(x, w, r)


def _mm_res_proj_kernel(x_ref, w_ref, r_ref, pw_ref, pb_ref, o_ref):
    t = r_ref[...] + jnp.dot(x_ref[...].astype(w_ref.dtype), w_ref[...],
                             preferred_element_type=jnp.float32)
    o_ref[...] = jnp.dot(t.astype(pw_ref.dtype), pw_ref[...],
                         preferred_element_type=jnp.float32) + pb_ref[...]


def _mm_res_proj(x, w, r, pw, pb):
    m, k = x.shape
    n = w.shape[1]
    n2 = pw.shape[1]
    bm = min(BM, m)
    return pl.pallas_call(
        _mm_res_proj_kernel,
        grid=(m // bm,),
        in_specs=[
            pl.BlockSpec((bm, k), lambda i: (i, 0)),
            pl.BlockSpec((k, n), lambda i: (0, 0)),
            pl.BlockSpec((bm, n), lambda i: (i, 0)),
            pl.BlockSpec((n, n2), lambda i: (0, 0)),
            pl.BlockSpec((1, n2), lambda i: (0, 0)),
        ],
        out_specs=pl.BlockSpec((bm, n2), lambda i: (i, 0)),
        out_shape=jax.ShapeDtypeStruct((m, n2), jnp.float32),
    )(x, w, r, pw, pb.reshape(1, n2))


def _mm_bias_res_kernel(x_ref, w_ref, b_ref, r_ref, o_ref):
    o_ref[...] = r_ref[...] + b_ref[...] + jnp.dot(
        x_ref[...].astype(w_ref.dtype), w_ref[...],
        preferred_element_type=jnp.float32)


def _mm_bias_res(x, w, b, r, x_wrap=None):
    k = x.shape[1]
    m = r.shape[0]
    n = w.shape[1]
    bm = min(BM, m)
    if x_wrap is None:
        x_spec = pl.BlockSpec((bm, k), lambda i: (i, 0))
    else:
        x_spec = pl.BlockSpec((bm, k), lambda i, w_=x_wrap: (i % w_, 0))
    return pl.pallas_call(
        _mm_bias_res_kernel,
        grid=(m // bm,),
        in_specs=[
            x_spec,
            pl.BlockSpec((k, n), lambda i: (0, 0)),
            pl.BlockSpec((1, n), lambda i: (0, 0)),
            pl.BlockSpec((bm, n), lambda i: (i, 0)),
        ],
        out_specs=pl.BlockSpec((bm, n), lambda i: (i, 0)),
        out_shape=jax.ShapeDtypeStruct((m, n), jnp.float32),
    )(x, w, b.reshape(1, n), r)


def _w2_gate_kernel(u_ref, w_ref, r_ref, g_ref, w1_ref, b1_ref, w2_ref,
                    b2_ref, o_ref):
    scan = r_ref[...] + jnp.dot(u_ref[...].astype(w_ref.dtype), w_ref[...],
                                preferred_element_type=jnp.float32)
    hln = _ln_f32(scan, g_ref[...])
    hmid = jnp.maximum(
        jnp.dot(hln, w1_ref[...], preferred_element_type=jnp.float32)
        + b1_ref[...], 0.0)
    o_ref[...] = jnp.dot(hmid, w2_ref[...],
                         preferred_element_type=jnp.float32) + b2_ref[...]


def _w2_gate(u, w, r, g, w1, b1, w2, b2):
    m, k = u.shape
    n = w.shape[1]
    kh = w1.shape[1]
    bm = min(BM, m)
    return pl.pallas_call(
        _w2_gate_kernel,
        grid=(m // bm,),
        in_specs=[
            pl.BlockSpec((bm, k), lambda i: (i, 0)),
            pl.BlockSpec((k, n), lambda i: (0, 0)),
            pl.BlockSpec((bm, n), lambda i: (i, 0)),
            pl.BlockSpec((1, n), lambda i: (0, 0)),
            pl.BlockSpec((n, kh), lambda i: (0, 0)),
            pl.BlockSpec((1, kh), lambda i: (0, 0)),
            pl.BlockSpec((kh, 1), lambda i: (0, 0)),
            pl.BlockSpec((1, 1), lambda i: (0, 0)),
        ],
        out_specs=pl.BlockSpec((bm, 1), lambda i: (i, 0)),
        out_shape=jax.ShapeDtypeStruct((m, 1), jnp.float32),
    )(u, w, r, g.reshape(1, n), w1, b1.reshape(1, kh), w2, b2.reshape(1, 1))


def _mask_kernel(gc_ref, gr_ref, o_ref, *, bt, s, si):
    i = pl.program_id(1)
    gi = gc_ref[0]
    gj = gr_ref[0]
    gt = (gj > gi).astype(jnp.float32)
    row = jax.lax.broadcasted_iota(jnp.int32, (bt, s), 0) + i * bt
    col = jax.lax.broadcasted_iota(jnp.int32, (bt, s), 1)
    eq = jnp.logical_and(gj == gi, col < row).astype(jnp.float32)
    rank = jnp.sum(gt + eq, axis=1, keepdims=True)
    o_ref[0] = (rank < si).astype(jnp.float32)


def _topk_mask(gate_b, si):
    b, s = gate_b.shape
    bt = min(BM, s)
    gc = gate_b.reshape(b, s, 1)
    gr = gate_b.reshape(b, 1, s)
    maskc = pl.pallas_call(
        functools.partial(_mask_kernel, bt=bt, s=s, si=si),
        grid=(b, s // bt),
        in_specs=[
            pl.BlockSpec((1, bt, 1), lambda b_, i: (b_, i, 0)),
            pl.BlockSpec((1, 1, s), lambda b_, i: (b_, 0, 0)),
        ],
        out_specs=pl.BlockSpec((1, bt, 1), lambda b_, i: (b_, i, 0)),
        out_shape=jax.ShapeDtypeStruct((b, s, 1), jnp.float32),
    )(gc, gr)
    return maskc.reshape(b * s, 1)


def _up_ae_kernel(u_ref, w2_ref, r_ref, uw_ref, ub_ref, m_ref, ew_ref,
                  eb_ref, dw_ref, db_ref, sc_ref, acc_ref):
    i = pl.program_id(0)

    @pl.when(i == 0)
    def _init():
        acc_ref[...] = jnp.zeros_like(acc_ref)

    g2d = r_ref[...] + jnp.dot(u_ref[...], w2_ref[...],
                               preferred_element_type=jnp.float32)
    up = jnp.dot(g2d.astype(uw_ref.dtype), uw_ref[...],
                 preferred_element_type=jnp.float32) + ub_ref[...]
    mk = m_ref[...]
    z = jnp.dot(up.astype(ew_ref.dtype), ew_ref[...],
                preferred_element_type=jnp.float32) + eb_ref[...]
    hard = (z > 0).astype(dw_ref.dtype)
    ae = jnp.dot(hard, dw_ref[...], preferred_element_type=jnp.float32) \
        + db_ref[...]
    dlt = up - ae
    acc_ref[...] += jnp.sum(dlt * dlt * mk, axis=(0, 1), keepdims=True)
    sc_ref[...] = up * mk


def _up_ae_scatter(u, w2, r, uw, ub, maskcol, ew, eb, dw, db):
    m, kk = u.shape
    d = w2.shape[1]
    di = uw.shape[1]
    code = ew.shape[1]
    bm = min(BM, m)
    return pl.pallas_call(
        _up_ae_kernel,
        grid=(m // bm,),
        in_specs=[
            pl.BlockSpec((bm, kk), lambda i: (i, 0)),
            pl.BlockSpec((kk, d), lambda i: (0, 0)),
            pl.BlockSpec((bm, d), lambda i: (i, 0)),
            pl.BlockSpec((d, di), lambda i: (0, 0)),
            pl.BlockSpec((1, di), lambda i: (0, 0)),
            pl.BlockSpec((bm, 1), lambda i: (i, 0)),
            pl.BlockSpec((di, code), lambda i: (0, 0)),
            pl.BlockSpec((1, code), lambda i: (0, 0)),
            pl.BlockSpec((code, di), lambda i: (0, 0)),
            pl.BlockSpec((1, di), lambda i: (0, 0)),
        ],
        out_specs=[
            pl.BlockSpec((bm, di), lambda i: (i, 0)),
            pl.BlockSpec((1, 1), lambda i: (0, 0)),
        ],
        out_shape=[
            jax.ShapeDtypeStruct((m, di), jnp.float32),
            jax.ShapeDtypeStruct((1, 1), jnp.float32),
        ],
    )(u, w2, r, uw, ub.reshape(1, di), maskcol, ew, eb.reshape(1, code),
      dw, db.reshape(1, di))


def _bf(w):
    return w.astype(_BF)


def _qkv_weights(w, d, groups):
    dh = d // H
    return _bf(w.reshape(d, groups, H, dh).transpose(1, 2, 0, 3)
               .reshape(groups * H, d, dh))


def _wo_weights(w, d):
    dh = d // H
    return _bf(w.reshape(H, dh, w.shape[1]))


def _block2d(x2d, p, causal, b, s):
    d = x2d.shape[1]
    qkv = _qkv(x2d, p["ln1"], _qkv_weights(p["Wqkv"], d, 3))
    if causal:
        ah = _attn_causal_hm(qkv, qkv, H, 2 * H, b, s)
    else:
        ah = _attn_hm(qkv, qkv, H, 2 * H, False, b, s)
    x2 = _wo_res(ah, _wo_weights(p["Wo"], d), x2d)
    u = _lnmm(x2, p["ln2"], _bf(p["W1"]), act="gelu", out_dtype=_BF)
    return x2, u


def _block(x2d, p, causal, b, s):
    x2, u = _block2d(x2d, p, causal, b, s)
    return _mm_res(u, _bf(p["W2"]), x2)


def _combine_parts(kv2d, q2d, p, b, s, q_shared=False):
    d = q2d.shape[1]
    nbs = s // min(BM, s)
    hq = _qkv(q2d, p["lnq"], _qkv_weights(p["Wq"], d, 1))
    hkv = _qkv(kv2d, p["lnkv"], _qkv_weights(p["Wkv"], d, 2))
    ah = _attn_causal_hm(hq, hkv, 0, H, b, s, q_shared=q_shared)
    x2 = _wo_res(ah, _wo_weights(p["Wo"], d), q2d,
                 r_wrap=nbs if q_shared else None)
    u = _lnmm(x2, p["ln2"], _bf(p["W1"]), act="gelu", out_dtype=_BF)
    return x2, u


def _combine(kv2d, q2d, p, b, s):
    x2, u = _combine_parts(kv2d, q2d, p, b, s)
    return _mm_res(u, _bf(p["W2"]), x2)


def kernel(x, params):
    p = params
    b, s, d = x.shape
    di = p["up_proj_w"].shape[1]
    si = s // 4
    m = b * s

    x2d = x.reshape(m, d)

    x2u, uu = _block2d(x2d, p["up_scan"], False, b, s)
    gate = _w2_gate(uu, _bf(p["up_scan"]["W2"]), x2u, p["up_norm_g"],
                    p["up_gate_w1"], p["up_gate_b1"],
                    p["up_gate_w2"], p["up_gate_b2"])
    maskcol = _topk_mask(gate.reshape(b, s), si)

    x2q, uq = _block2d(x2d, p["query_block"], True, b, s)
    scattered, acc = _up_ae_scatter(
        uq, _bf(p["query_block"]["W2"]), x2q,
        _bf(p["up_proj_w"]), p["up_proj_b"], maskcol,
        _bf(p["ae_enc_w"]), p["ae_enc_b"], _bf(p["ae_dec_w"]),
        p["ae_dec_b"])
    aux = acc[0, 0] / jnp.float32(b * si * di)

    pos = p["pos_emb_w"] + p["pos_emb_b"][None, :]
    nbs = s // min(BM, s)

    x2c, uc = _combine_parts(scattered, pos, p["down_scatter"], b, s,
                             q_shared=True)
    pdsp = _mm_res_proj(uc, _bf(p["down_scatter"]["W2"]), x2c,
                        _bf(p["down_proj_w"]), p["down_proj_b"])

    q2 = _mm_bias_res(pos, _bf(p["down_proj_w"]), p["down_proj_b"], x2d,
                      x_wrap=nbs)
    p_down = _combine(pdsp, q2, p["down_scatter2"], b, s)

    out = _block(p_down, p["down_scan"], False, b, s)
    return out.reshape(b, s, d), aux
